# baseline (speedup 1.0000x reference)
import functools

import jax
import jax.numpy as jnp
from jax import lax
from jax.experimental import pallas as pl
from jax.experimental.pallas import tpu as pltpu
from jax.experimental.pallas import tpu_sc as plsc

BATCH = 16384
EMBED = 32
LANES = 16
BLOCK = 128
ROWS_PER_BLOCK = BLOCK // EMBED
CHUNK = 256


@functools.lru_cache(maxsize=None)
def _make_kernel(num_cores: int, num_subcores: int):
    num_workers = num_cores * num_subcores
    b_per_w = BATCH // num_workers
    num_chunks = b_per_w // CHUNK
    mesh = plsc.VectorSubcoreMesh(core_axis_name="c", subcore_axis_name="s")

    @functools.partial(
        pl.kernel,
        out_type=jax.ShapeDtypeStruct((BATCH,), jnp.float32),
        mesh=mesh,
        compiler_params=pltpu.CompilerParams(needs_layout_passes=False,
                                             use_tc_tiling_on_sc=True),
        scratch_types=[
            pltpu.VMEM((b_per_w,), jnp.int32),
            pltpu.VMEM((b_per_w,), jnp.int32),
            pltpu.VMEM((b_per_w,), jnp.int32),
            pltpu.VMEM((b_per_w,), jnp.int32),
            pltpu.VMEM((CHUNK, BLOCK), jnp.float32),
            pltpu.VMEM((CHUNK, BLOCK), jnp.float32),
            pltpu.VMEM((b_per_w,), jnp.float32),
            pltpu.SemaphoreType.DMA,
        ],
    )
    def sc_kernel(uids_hbm, iids_hbm, utab_hbm, itab_hbm, out_hbm,
                  uidx_v, iidx_v, ubid_v, ibid_v, ublk_v, iblk_v, out_v, sem):
        wid = lax.axis_index("s") * num_cores + lax.axis_index("c")
        base = wid * b_per_w
        pltpu.sync_copy(uids_hbm.at[pl.ds(base, b_per_w)], uidx_v)
        pltpu.sync_copy(iids_hbm.at[pl.ds(base, b_per_w)], iidx_v)

        def bid_body(k, carry):
            sl = pl.ds(k * LANES, LANES)
            ubid_v[sl] = lax.shift_right_logical(uidx_v[sl], 2)
            ibid_v[sl] = lax.shift_right_logical(iidx_v[sl], 2)
            return carry

        lax.fori_loop(0, b_per_w // LANES, bid_body, 0)

        lanes = lax.iota(jnp.int32, LANES)
        for c in range(num_chunks):
            cu = pltpu.async_copy(
                utab_hbm.at[ubid_v.at[pl.ds(c * CHUNK, CHUNK)]], ublk_v, sem)
            ci = pltpu.async_copy(
                itab_hbm.at[ibid_v.at[pl.ds(c * CHUNK, CHUNK)]], iblk_v, sem)
            cu.wait()
            ci.wait()

            def body(g, carry):
                r0 = c * CHUNK + g * LANES
                rows = lanes + g * LANES
                uoff = (uidx_v[pl.ds(r0, LANES)] & 3) * EMBED
                ioff = (iidx_v[pl.ds(r0, LANES)] & 3) * EMBED
                acc = jnp.zeros((LANES,), jnp.float32)
                for d in range(EMBED):
                    ucol = plsc.load_gather(ublk_v, [rows, uoff + d])
                    icol = plsc.load_gather(iblk_v, [rows, ioff + d])
                    acc = acc + ucol * icol
                out_v[pl.ds(r0, LANES)] = acc
                return carry

            lax.fori_loop(0, CHUNK // LANES, body, 0)

        pltpu.sync_copy(out_v, out_hbm.at[pl.ds(base, b_per_w)])

    return sc_kernel


def kernel(user_ids, item_ids, user_table, item_table):
    info = plsc.get_sparse_core_info()
    sc_kernel = _make_kernel(info.num_cores, info.num_subcores)
    utab = user_table.reshape(user_table.shape[0] // ROWS_PER_BLOCK, BLOCK)
    itab = item_table.reshape(item_table.shape[0] // ROWS_PER_BLOCK, BLOCK)
    return sc_kernel(user_ids.astype(jnp.int32), item_ids.astype(jnp.int32),
                     utab, itab)
